# Initial kernel scaffold; baseline (speedup 1.0000x reference)
import jax
import jax.numpy as jnp
from jax.experimental import pallas as pl


def kernel(x, edge_index, W, b):
    raise NotImplementedError("write your pallas kernel here")


# speedup vs baseline: 21.5578x; 21.5578x over previous
import jax
import jax.numpy as jnp
from jax import lax
from jax.experimental import pallas as pl
from jax.experimental.pallas import tpu as pltpu
from jax.experimental.pallas import tpu_sc as plsc

N_NODES = 10000
D = 128
NC, NS, L = 2, 16, 16
NW = NC * NS
CHUNK = 128
NN = 10240
RPT = NN // NS

_mesh = plsc.VectorSubcoreMesh(
    core_axis_name="c", subcore_axis_name="s", num_cores=NC, num_subcores=NS
)


def _deg_body(dst_hbm, degp_hbm, dst_v, deg_v):
    c = lax.axis_index("c")
    s = lax.axis_index("s")
    wid = c * NS + s
    pltpu.sync_copy(dst_hbm.at[wid], dst_v)

    def zero_body(i, carry):
        deg_v[pl.ds(pl.multiple_of(i * L, L), L)] = jnp.zeros((L,), jnp.float32)
        return carry

    lax.fori_loop(0, NN // L, zero_body, 0)

    ones = jnp.ones((L,), jnp.float32)
    n_chunks = dst_v.shape[0]

    def chunk_body(ch, carry):
        for j in range(CHUNK // L):
            idx = dst_v[ch, pl.ds(j * L, L)]
            plsc.addupdate_scatter(deg_v, [idx], ones)
        return carry

    lax.fori_loop(0, n_chunks, chunk_body, 0)
    pltpu.sync_copy(deg_v, degp_hbm.at[wid])


def _msg_body(src_hbm, dst_hbm, y_hbm, p_hbm, src_v, dst_v, rows_v, acc_sh, sem):
    c = lax.axis_index("c")
    s = lax.axis_index("s")
    wid = c * NS + s
    pltpu.sync_copy(src_hbm.at[wid], src_v)
    pltpu.sync_copy(dst_hbm.at[wid], dst_v)

    def zero_rows(r, carry):
        for j in range(D // L):
            rows_v[r, pl.ds(j * L, L)] = jnp.zeros((L,), jnp.float32)
        return carry

    lax.fori_loop(0, CHUNK, zero_rows, 0)
    base = pl.multiple_of(s * RPT, CHUNK)
    for k in range(RPT // CHUNK):
        pltpu.sync_copy(rows_v, acc_sh.at[pl.ds(base + k * CHUNK, CHUNK)])
    plsc.subcore_barrier()

    n_chunks = src_v.shape[0]

    def chunk_body(ch, carry):
        pltpu.async_copy(y_hbm.at[src_v.at[ch]], rows_v, sem).wait()
        pltpu.sync_copy(rows_v, acc_sh.at[dst_v.at[ch]], add=True)
        return carry

    lax.fori_loop(0, n_chunks, chunk_body, 0)
    plsc.subcore_barrier()
    for k in range(RPT // CHUNK):
        pltpu.sync_copy(
            acc_sh.at[pl.ds(base + k * CHUNK, CHUNK)],
            p_hbm.at[c].at[pl.ds(base + k * CHUNK, CHUNK)],
        )


def _dense1_body(x_ref, w_ref, degpt_ref, y_ref):
    deg = jnp.sum(degpt_ref[...], axis=1, keepdims=True) + 1.0
    dis = lax.rsqrt(deg)
    xlin = jnp.dot(x_ref[...], w_ref[...], preferred_element_type=jnp.float32)
    y = xlin * dis[:N_NODES]
    y_ref[...] = jnp.concatenate(
        [y, jnp.zeros((NN - N_NODES, D), jnp.float32)], axis=0
    )


def _dense2_body(degpt_ref, y_ref, p_ref, b_ref, out_ref):
    deg = jnp.sum(degpt_ref[...], axis=1, keepdims=True) + 1.0
    dis = lax.rsqrt(deg)
    tot = p_ref[0, :N_NODES, :] + p_ref[1, :N_NODES, :] + y_ref[:N_NODES, :]
    out_ref[...] = jnp.maximum(dis[:N_NODES] * tot + b_ref[...][None, :], 0.0)


def kernel(x, edge_index, W, b):
    E = edge_index.shape[1]
    ept = -(-E // (NW * CHUNK)) * CHUNK
    epad = NW * ept - E
    src = edge_index[0].astype(jnp.int32)
    dst = edge_index[1].astype(jnp.int32)
    fill = jnp.full((epad,), N_NODES, jnp.int32)
    src_p = jnp.concatenate([src, fill]).reshape(NW, ept // CHUNK, CHUNK)
    dst_p = jnp.concatenate([dst, fill]).reshape(NW, ept // CHUNK, CHUNK)

    deg_call = pl.kernel(
        _deg_body,
        out_type=jax.ShapeDtypeStruct((NW, NN), jnp.float32),
        mesh=_mesh,
        scratch_types=[
            pltpu.VMEM((ept // CHUNK, CHUNK), jnp.int32),
            pltpu.VMEM((NN,), jnp.float32),
        ],
        compiler_params=pltpu.CompilerParams(needs_layout_passes=False),
    )
    degp = deg_call(dst_p)
    degpt = degp.T

    y = pl.pallas_call(
        _dense1_body,
        out_shape=jax.ShapeDtypeStruct((NN, D), jnp.float32),
    )(x, W, degpt)

    msg_call = pl.kernel(
        _msg_body,
        out_type=jax.ShapeDtypeStruct((NC, NN, D), jnp.float32),
        mesh=_mesh,
        scratch_types=[
            pltpu.VMEM((ept // CHUNK, CHUNK), jnp.int32),
            pltpu.VMEM((ept // CHUNK, CHUNK), jnp.int32),
            pltpu.VMEM((CHUNK, D), jnp.float32),
            pltpu.VMEM_SHARED((NN, D), jnp.float32),
            pltpu.SemaphoreType.DMA,
        ],
    )
    p = msg_call(src_p, dst_p, y)

    out = pl.pallas_call(
        _dense2_body,
        out_shape=jax.ShapeDtypeStruct((N_NODES, D), jnp.float32),
    )(degpt, y, p, b)
    return out
